# baseline (speedup 1.0000x reference)
import functools

import jax
import jax.numpy as jnp
from jax import lax
from jax.experimental import pallas as pl
from jax.experimental.pallas import tpu as pltpu
from jax.experimental.pallas import tpu_sc as plsc

NV = [16384, 4096, 1024, 256, 64]
SEQ = 12
NC = 2
NS = 16
NW = NC * NS
BSZ = 8


def _matmul_kernel(x_ref, w_ref, b_ref, o_ref):
    o_ref[...] = (
        jnp.dot(x_ref[...], w_ref[...], preferred_element_type=jnp.float32)
        + b_ref[...]
    )


def _matmul(x, w, b, block_m=512):
    m, k = x.shape
    n = w.shape[1]
    bm = min(block_m, m)
    return pl.pallas_call(
        _matmul_kernel,
        grid=(pl.cdiv(m, bm),),
        in_specs=[
            pl.BlockSpec((bm, k), lambda i: (i, 0)),
            pl.BlockSpec((k, n), lambda i: (0, 0)),
            pl.BlockSpec((n,), lambda i: (0,)),
        ],
        out_specs=pl.BlockSpec((bm, n), lambda i: (i, 0)),
        out_shape=jax.ShapeDtypeStruct((m, n), jnp.float32),
        compiler_params=pltpu.CompilerParams(
            dimension_semantics=("arbitrary",),
        ),
    )(x, w, b)


def _matmul_z(x, w2, co, block_m=512):
    m, k = x.shape
    bm = min(block_m, m)

    def kern(x_ref, w_ref, o_ref):
        res = jnp.dot(x_ref[...], w_ref[...],
                      preferred_element_type=jnp.float32)
        for s in range(SEQ):
            o_ref[s] = res[:, s * co:(s + 1) * co]

    z = pl.pallas_call(
        kern,
        grid=(pl.cdiv(m, bm),),
        in_specs=[
            pl.BlockSpec((bm, k), lambda i: (i, 0)),
            pl.BlockSpec((k, SEQ * co), lambda i: (0, 0)),
        ],
        out_specs=pl.BlockSpec((SEQ, bm, co), lambda i: (0, i, 0)),
        out_shape=jax.ShapeDtypeStruct((SEQ, m, co), jnp.float32),
        compiler_params=pltpu.CompilerParams(
            dimension_semantics=("arbitrary",),
        ),
    )(x, w2)
    return z.reshape(SEQ * m, co)


def _sc_mesh():
    return plsc.VectorSubcoreMesh(core_axis_name="c", subcore_axis_name="s")


_SC_PARAMS = pltpu.CompilerParams(use_tc_tiling_on_sc=False)


def _sc_pool(x2d, rvc, bounds, n_in, n_out, ch):
    nnz = rvc.shape[0] // 3
    bpc = BSZ // NC
    kc = 128
    rpw = n_out // NS
    kc3 = 3 * kc

    def body(x_hbm, rvc_hbm, bounds_hbm, out_hbm,
             y_v, g0, g1, rvc0, rvc1, rvk0, rvk1, cb0, cb1, bounds_v,
             sl0, sl1, sg0, sg1):
        c_idx = lax.axis_index("c")
        s_idx = lax.axis_index("s")

        pltpu.sync_copy(bounds_hbm, bounds_v)
        ks = bounds_v[pl.ds(s_idx, 16)][0]
        ke = bounds_v[pl.ds(s_idx + 1, 16)][0]
        cs = ks // kc
        ce = (ke + kc - 1) // kc
        lo = s_idx * rpw
        have = ks < ke

        zero16 = jnp.zeros((16,), jnp.float32)
        gs = (g0, g1)
        rvcs = (rvc0, rvc1)
        rvks = (rvk0, rvk1)
        cbs = (cb0, cb1)
        sls = (sl0, sl1)
        sgs = (sg0, sg1)

        def fire_lin(c, par):
            pltpu.async_copy(
                rvc_hbm.at[pl.ds(c * kc3, kc3)], rvcs[par], sls[par])

        def fire_gather(c, par, bg):
            pltpu.make_async_copy(
                rvc_hbm.at[pl.ds(0, kc3)], rvcs[par], sls[par]).wait()
            rvc_v, rvk_v, cb_v = rvcs[par], rvks[par], cbs[par]
            boff = bg * n_in
            for g in range(kc // 16):
                sl16 = pl.ds(g * 16, 16)
                rvk_v[sl16] = rvc_v[sl16]
                sl16b = pl.ds(kc + g * 16, 16)
                rvk_v[sl16b] = rvc_v[sl16b]
                cf = rvc_v[pl.ds(2 * kc + g * 16, 16)]
                cb_v[sl16] = cf.astype(jnp.int32) + boff
            pltpu.async_copy(x_hbm.at[cb_v], gs[par], sgs[par])

        def process(c, par):
            pltpu.make_async_copy(
                x_hbm.at[pl.ds(0, kc)], gs[par], sgs[par]).wait()
            g_v, rvk_v = gs[par], rvks[par]
            k0 = c * kc

            def gbody(g, _):
                rows16 = rvk_v[pl.ds(g * 16, 16)].astype(jnp.int32)
                vals16 = rvk_v[pl.ds(kc + g * 16, 16)]
                for t in range(16):
                    k = g * 16 + t
                    kg = k0 + k
                    r = rows16[t]
                    v = vals16[t]
                    valid = jnp.logical_and(kg >= ks, kg < ke)
                    vm = jnp.where(valid, v, 0.0)
                    r_l = jnp.clip(r - lo, 0, rpw - 1)
                    vsp = jnp.full((16,), vm, jnp.float32)
                    for j in range(ch // 16):
                        sl = pl.ds(j * 16, 16)
                        y_v[r_l, sl] = y_v[r_l, sl] + vsp * g_v[k, sl]
                return 0

            lax.fori_loop(0, kc // 16, gbody, 0)

        def batch_body(bl, _):
            bg = c_idx * bpc + bl

            def zrow(i, _):
                for j in range(ch // 16):
                    y_v[i, pl.ds(j * 16, 16)] = zero16
                return 0

            lax.fori_loop(0, rpw, zrow, 0)

            @pl.when(have)
            def _():
                fire_lin(cs, 0)

            def phase(i, par):
                c = cs + i

                @pl.when(c < ce)
                def _():
                    fire_gather(c, par, bg)

                @pl.when(c + 1 < ce)
                def _():
                    fire_lin(c + 1, 1 - par)

                @pl.when(jnp.logical_and(i >= 1, c - 1 < ce))
                def _():
                    process(c - 1, 1 - par)

            def body2(p, _):
                phase(2 * p, 0)
                phase(2 * p + 1, 1)
                return 0

            lax.fori_loop(0, (ce - cs + 2) // 2, body2, 0)
            pltpu.sync_copy(
                y_v, out_hbm.at[pl.ds(bg * n_out + lo, rpw)])
            return 0

        lax.fori_loop(0, bpc, batch_body, 0)

    fn = pl.kernel(
        body,
        out_type=jax.ShapeDtypeStruct((BSZ * n_out, ch), jnp.float32),
        mesh=_sc_mesh(),
        compiler_params=_SC_PARAMS,
        scratch_types=[
            pltpu.VMEM((rpw, ch), jnp.float32),
            pltpu.VMEM((kc, ch), jnp.float32),
            pltpu.VMEM((kc, ch), jnp.float32),
            pltpu.VMEM((kc3,), jnp.float32),
            pltpu.VMEM((kc3,), jnp.float32),
            pltpu.VMEM((2 * kc,), jnp.float32),
            pltpu.VMEM((2 * kc,), jnp.float32),
            pltpu.VMEM((kc,), jnp.int32),
            pltpu.VMEM((kc,), jnp.int32),
            pltpu.VMEM((32,), jnp.int32),
            pltpu.SemaphoreType.DMA,
            pltpu.SemaphoreType.DMA,
            pltpu.SemaphoreType.DMA,
            pltpu.SemaphoreType.DMA,
        ],
    )
    return fn(x2d, rvc, bounds)


def _sc_gsum(zf, idxg2, bias, n_rows_out, co, vchunk, elu):
    rows_per_w = n_rows_out // NW
    v = vchunk
    assert rows_per_w % v == 0
    nchunks = rows_per_w // v
    assert nchunks % 2 == 0
    vs = v * SEQ

    def body(z_hbm, idx_hbm, bias_hbm, out_hbm,
             idx_v, a0, a1, b_v, sg0, sg1, so0, so1):
        wid = lax.axis_index("s") * NC + lax.axis_index("c")
        row0 = wid * rows_per_w
        pltpu.sync_copy(bias_hbm, b_v)
        pltpu.sync_copy(
            idx_hbm.at[pl.ds(row0 * SEQ, rows_per_w * SEQ)], idx_v)

        zero16 = jnp.zeros((16,), jnp.float32)
        accs = (a0, a1)
        sgs = (sg0, sg1)
        sos = (so0, so1)

        def stage_fire(ci, par, first):
            acc, sg, so = accs[par], sgs[par], sos[par]
            if not first:
                pltpu.make_async_copy(
                    z_hbm.at[pl.ds(0, v)], acc, so).wait()

            def zrow(i, _):
                for j in range(co // 16):
                    acc[i, pl.ds(j * 16, 16)] = zero16
                return 0

            lax.fori_loop(0, v, zrow, 0)
            for s in range(SEQ):
                pltpu.async_copy(
                    z_hbm.at[idx_v.at[pl.ds((ci * SEQ + s) * v, v)]],
                    acc, sg, add=True)

        def stage_proc(ci, par):
            acc, sg, so = accs[par], sgs[par], sos[par]
            for _ in range(SEQ):
                pltpu.make_async_copy(
                    z_hbm.at[pl.ds(0, v)], acc, sg).wait()

            def vbody(vi, _):
                for j in range(co // 16):
                    sl = pl.ds(j * 16, 16)
                    r = acc[vi, sl] + b_v[sl]
                    if elu:
                        r = jnp.where(r > 0.0, r, jnp.exp(r) - 1.0)
                    acc[vi, sl] = r
                return 0

            lax.fori_loop(0, v, vbody, 0)
            pltpu.async_copy(
                acc, out_hbm.at[pl.ds(row0 + ci * v, v)], so)

        stage_fire(0, 0, True)
        stage_fire(1, 1, True)

        def body2(p, _):
            c = 2 * p
            stage_proc(c, 0)

            @pl.when(c + 2 < nchunks)
            def _():
                stage_fire(c + 2, 0, False)

            stage_proc(c + 1, 1)

            @pl.when(c + 3 < nchunks)
            def _():
                stage_fire(c + 3, 1, False)

            return 0

        lax.fori_loop(0, nchunks // 2, body2, 0)
        pltpu.make_async_copy(z_hbm.at[pl.ds(0, v)], a0, so0).wait()
        pltpu.make_async_copy(z_hbm.at[pl.ds(0, v)], a1, so1).wait()

    fn = pl.kernel(
        body,
        out_type=jax.ShapeDtypeStruct((n_rows_out, co), jnp.float32),
        mesh=_sc_mesh(),
        compiler_params=_SC_PARAMS,
        scratch_types=[
            pltpu.VMEM((rows_per_w * SEQ,), jnp.int32),
            pltpu.VMEM((v, co), jnp.float32),
            pltpu.VMEM((v, co), jnp.float32),
            pltpu.VMEM((co,), jnp.float32),
            pltpu.SemaphoreType.DMA,
            pltpu.SemaphoreType.DMA,
            pltpu.SemaphoreType.DMA,
            pltpu.SemaphoreType.DMA,
        ],
    )
    return fn(zf, idxg2, bias)


def _conv_w2(w, ci, co):
    return w.reshape(SEQ, ci, co).transpose(1, 0, 2).reshape(ci, SEQ * co)


def _gather_idx(idx, n_in, v):
    b_off = (jnp.arange(BSZ, dtype=jnp.int32) * n_in)[:, None, None]
    s_off = (jnp.arange(SEQ, dtype=jnp.int32) * (BSZ * n_in))[None, None, :]
    g = b_off + idx[None, :, :] + s_off
    g = g.reshape(-1, SEQ)
    return g.reshape(-1, v, SEQ).transpose(0, 2, 1).reshape(-1)


def kernel(x, spiral_idx0, spiral_idx1, spiral_idx2, spiral_idx3,
           up_rows0, up_cols0, up_vals0, up_rows1, up_cols1, up_vals1,
           up_rows2, up_cols2, up_vals2, up_rows3, up_cols3, up_vals3,
           fc_W, fc_b, Wd0, bd0, Wd1, bd1, Wd2, bd2, Wd3, bd3, Wout, bout):
    levels = [
        (NV[4], NV[3], 256, 256, spiral_idx3, up_rows3, up_cols3, up_vals3, Wd0, bd0, 32),
        (NV[3], NV[2], 256, 128, spiral_idx2, up_rows2, up_cols2, up_vals2, Wd1, bd1, 128),
        (NV[2], NV[1], 128, 64, spiral_idx1, up_rows1, up_cols1, up_vals1, Wd2, bd2, 128),
        (NV[1], NV[0], 64, 64, spiral_idx0, up_rows0, up_cols0, up_vals0, Wd3, bd3, 128),
    ]

    h = _matmul(x, fc_W, fc_b, block_m=8).reshape(BSZ * NV[4], 256)

    b_ar = jnp.arange(BSZ, dtype=jnp.int32)
    for n_in, n_out, ci, co, sp_idx, rows, cols, vals, W, bb, vch in levels:
        nnz = rows.shape[0]
        nnz_p = -(-nnz // 128) * 128
        pad = nnz_p - nnz
        if pad:
            rows = jnp.concatenate(
                [rows, jnp.full((pad,), n_out - 1, jnp.int32)])
            cols = jnp.concatenate([cols, jnp.zeros((pad,), jnp.int32)])
            vals = jnp.concatenate([vals, jnp.zeros((pad,), jnp.float32)])
        rpw = n_out // NS
        bounds = jnp.searchsorted(
            rows, jnp.arange(NS + 1, dtype=jnp.int32) * rpw).astype(jnp.int32)
        bounds = jnp.concatenate(
            [bounds, jnp.zeros((32 - NS - 1,), jnp.int32)])
        rvc = jnp.stack([
            rows.astype(jnp.float32).reshape(-1, 128),
            vals.reshape(-1, 128),
            cols.astype(jnp.float32).reshape(-1, 128),
        ], axis=1).reshape(-1)
        y = _sc_pool(h, rvc, bounds, n_in, n_out, ci)
        zf = _matmul_z(y, _conv_w2(W, ci, co), co)
        idxg2 = _gather_idx(sp_idx, n_out, vch)
        h = _sc_gsum(zf, idxg2, bb, BSZ * n_out, co, vch, elu=True)

    co_p = 16
    w2o = jnp.zeros((64, SEQ * co_p), jnp.float32)
    wr = Wout.reshape(SEQ, 64, 3).transpose(1, 0, 2)
    w2o = w2o.reshape(64, SEQ, co_p).at[:, :, :3].set(wr).reshape(64, SEQ * co_p)
    bo = jnp.zeros((co_p,), jnp.float32).at[:3].set(bout)
    zf = _matmul_z(h, w2o, co_p)
    idxg2 = _gather_idx(spiral_idx0, NV[0], 128)
    out = _sc_gsum(zf, idxg2, bo, BSZ * NV[0], co_p, 128, elu=False)
    return out[:, :3].reshape(BSZ, NV[0], 3)


# speedup vs baseline: 1.3161x; 1.3161x over previous
import functools

import jax
import jax.numpy as jnp
from jax import lax
from jax.experimental import pallas as pl
from jax.experimental.pallas import tpu as pltpu
from jax.experimental.pallas import tpu_sc as plsc

NV = [16384, 4096, 1024, 256, 64]
SEQ = 12
NC = 2
NS = 16
NW = NC * NS
BSZ = 8


def _matmul_kernel(x_ref, w_ref, b_ref, o_ref):
    o_ref[...] = (
        jnp.dot(x_ref[...], w_ref[...], preferred_element_type=jnp.float32)
        + b_ref[...]
    )


def _matmul(x, w, b, block_m=512):
    m, k = x.shape
    n = w.shape[1]
    bm = min(block_m, m)
    return pl.pallas_call(
        _matmul_kernel,
        grid=(pl.cdiv(m, bm),),
        in_specs=[
            pl.BlockSpec((bm, k), lambda i: (i, 0)),
            pl.BlockSpec((k, n), lambda i: (0, 0)),
            pl.BlockSpec((n,), lambda i: (0,)),
        ],
        out_specs=pl.BlockSpec((bm, n), lambda i: (i, 0)),
        out_shape=jax.ShapeDtypeStruct((m, n), jnp.float32),
        compiler_params=pltpu.CompilerParams(
            dimension_semantics=("arbitrary",),
        ),
    )(x, w, b)


def _matmul_z(x, w2, co, block_m=512):
    m, k = x.shape
    bm = min(block_m, m)

    def kern(x_ref, w_ref, o_ref):
        o_ref[...] = jnp.dot(x_ref[...], w_ref[...],
                             preferred_element_type=jnp.float32)

    z = pl.pallas_call(
        kern,
        grid=(pl.cdiv(m, bm),),
        in_specs=[
            pl.BlockSpec((bm, k), lambda i: (i, 0)),
            pl.BlockSpec((k, SEQ * co), lambda i: (0, 0)),
        ],
        out_specs=pl.BlockSpec((bm, SEQ * co), lambda i: (i, 0)),
        out_shape=jax.ShapeDtypeStruct((m, SEQ * co), jnp.float32),
        compiler_params=pltpu.CompilerParams(
            dimension_semantics=("arbitrary",),
        ),
    )(x.astype(jnp.bfloat16), w2.astype(jnp.bfloat16))
    return z.reshape(m * SEQ, co)


def _sc_mesh():
    return plsc.VectorSubcoreMesh(core_axis_name="c", subcore_axis_name="s")


_SC_PARAMS = pltpu.CompilerParams(use_tc_tiling_on_sc=False)


def _sc_pool(x2d, rvc, bounds, n_in, n_out, ch):
    nnz = rvc.shape[0] // 3
    bpc = BSZ // NC
    kc = 128
    rpw = n_out // NS
    kc3 = 3 * kc

    def body(x_hbm, rvc_hbm, bounds_hbm, out_hbm,
             y_v, g0, g1, rvc0, rvc1, rvk0, rvk1, cb0, cb1, bounds_v,
             sl0, sl1, sg0, sg1):
        c_idx = lax.axis_index("c")
        s_idx = lax.axis_index("s")

        pltpu.sync_copy(bounds_hbm, bounds_v)
        ks = bounds_v[pl.ds(s_idx, 16)][0]
        ke = bounds_v[pl.ds(s_idx + 1, 16)][0]
        cs = ks // kc
        ce = (ke + kc - 1) // kc
        lo = s_idx * rpw
        have = ks < ke

        zero16 = jnp.zeros((16,), jnp.float32)
        gs = (g0, g1)
        rvcs = (rvc0, rvc1)
        rvks = (rvk0, rvk1)
        cbs = (cb0, cb1)
        sls = (sl0, sl1)
        sgs = (sg0, sg1)

        def fire_lin(c, par):
            pltpu.async_copy(
                rvc_hbm.at[pl.ds(c * kc3, kc3)], rvcs[par], sls[par])

        def fire_gather(c, par, bg):
            pltpu.make_async_copy(
                rvc_hbm.at[pl.ds(0, kc3)], rvcs[par], sls[par]).wait()
            rvc_v, rvk_v, cb_v = rvcs[par], rvks[par], cbs[par]
            boff = bg * n_in
            for g in range(kc // 16):
                sl16 = pl.ds(g * 16, 16)
                rvk_v[sl16] = rvc_v[sl16]
                sl16b = pl.ds(kc + g * 16, 16)
                rvk_v[sl16b] = rvc_v[sl16b]
                cf = rvc_v[pl.ds(2 * kc + g * 16, 16)]
                cb_v[sl16] = cf.astype(jnp.int32) + boff
            pltpu.async_copy(x_hbm.at[cb_v], gs[par], sgs[par])

        def process(c, par):
            pltpu.make_async_copy(
                x_hbm.at[pl.ds(0, kc)], gs[par], sgs[par]).wait()
            g_v, rvk_v = gs[par], rvks[par]
            k0 = c * kc

            def gbody(g, _):
                rows16 = rvk_v[pl.ds(g * 16, 16)].astype(jnp.int32)
                vals16 = rvk_v[pl.ds(kc + g * 16, 16)]
                for t in range(16):
                    k = g * 16 + t
                    kg = k0 + k
                    r = rows16[t]
                    v = vals16[t]
                    valid = jnp.logical_and(kg >= ks, kg < ke)
                    vm = jnp.where(valid, v, 0.0)
                    r_l = jnp.clip(r - lo, 0, rpw - 1)
                    vsp = jnp.full((16,), vm, jnp.float32)
                    for j in range(ch // 16):
                        sl = pl.ds(j * 16, 16)
                        y_v[r_l, sl] = y_v[r_l, sl] + vsp * g_v[k, sl]
                return 0

            lax.fori_loop(0, kc // 16, gbody, 0)

        def batch_body(bl, _):
            bg = c_idx * bpc + bl

            def zrow(i, _):
                for j in range(ch // 16):
                    y_v[i, pl.ds(j * 16, 16)] = zero16
                return 0

            lax.fori_loop(0, rpw, zrow, 0)

            @pl.when(have)
            def _():
                fire_lin(cs, 0)

            def phase(i, par):
                c = cs + i

                @pl.when(c < ce)
                def _():
                    fire_gather(c, par, bg)

                @pl.when(c + 1 < ce)
                def _():
                    fire_lin(c + 1, 1 - par)

                @pl.when(jnp.logical_and(i >= 1, c - 1 < ce))
                def _():
                    process(c - 1, 1 - par)

            def body2(p, _):
                phase(2 * p, 0)
                phase(2 * p + 1, 1)
                return 0

            lax.fori_loop(0, (ce - cs + 2) // 2, body2, 0)
            pltpu.sync_copy(
                y_v, out_hbm.at[pl.ds(bg * n_out + lo, rpw)])
            return 0

        lax.fori_loop(0, bpc, batch_body, 0)

    fn = pl.kernel(
        body,
        out_type=jax.ShapeDtypeStruct((BSZ * n_out, ch), jnp.float32),
        mesh=_sc_mesh(),
        compiler_params=_SC_PARAMS,
        scratch_types=[
            pltpu.VMEM((rpw, ch), jnp.float32),
            pltpu.VMEM((kc, ch), jnp.float32),
            pltpu.VMEM((kc, ch), jnp.float32),
            pltpu.VMEM((kc3,), jnp.float32),
            pltpu.VMEM((kc3,), jnp.float32),
            pltpu.VMEM((2 * kc,), jnp.float32),
            pltpu.VMEM((2 * kc,), jnp.float32),
            pltpu.VMEM((kc,), jnp.int32),
            pltpu.VMEM((kc,), jnp.int32),
            pltpu.VMEM((32,), jnp.int32),
            pltpu.SemaphoreType.DMA,
            pltpu.SemaphoreType.DMA,
            pltpu.SemaphoreType.DMA,
            pltpu.SemaphoreType.DMA,
        ],
    )
    return fn(x2d, rvc, bounds)


def _sc_gsum(zf, idxg2, bias, n_rows_out, co, vchunk, elu):
    rows_per_w = n_rows_out // NW
    v = vchunk
    assert rows_per_w % v == 0
    nchunks = rows_per_w // v
    assert nchunks % 2 == 0
    vs = v * SEQ

    def body(z_hbm, idx_hbm, bias_hbm, out_hbm,
             idx_v, a0, a1, b_v, sg0, sg1, so0, so1):
        wid = lax.axis_index("s") * NC + lax.axis_index("c")
        row0 = wid * rows_per_w
        pltpu.sync_copy(bias_hbm, b_v)
        pltpu.sync_copy(
            idx_hbm.at[pl.ds(row0 * SEQ, rows_per_w * SEQ)], idx_v)

        zero16 = jnp.zeros((16,), jnp.float32)
        accs = (a0, a1)
        sgs = (sg0, sg1)
        sos = (so0, so1)

        def stage_fire(ci, par, first):
            acc, sg, so = accs[par], sgs[par], sos[par]
            if not first:
                pltpu.make_async_copy(
                    z_hbm.at[pl.ds(0, v)], acc, so).wait()

            def zrow(i, _):
                for j in range(co // 16):
                    acc[i, pl.ds(j * 16, 16)] = zero16
                return 0

            lax.fori_loop(0, v, zrow, 0)
            for s in range(SEQ):
                pltpu.async_copy(
                    z_hbm.at[idx_v.at[pl.ds((ci * SEQ + s) * v, v)]],
                    acc, sg, add=True)

        def stage_proc(ci, par):
            acc, sg, so = accs[par], sgs[par], sos[par]
            for _ in range(SEQ):
                pltpu.make_async_copy(
                    z_hbm.at[pl.ds(0, v)], acc, sg).wait()

            def vbody(vi, _):
                for j in range(co // 16):
                    sl = pl.ds(j * 16, 16)
                    r = acc[vi, sl] + b_v[sl]
                    if elu:
                        r = jnp.where(r > 0.0, r, jnp.exp(r) - 1.0)
                    acc[vi, sl] = r
                return 0

            lax.fori_loop(0, v, vbody, 0)
            pltpu.async_copy(
                acc, out_hbm.at[pl.ds(row0 + ci * v, v)], so)

        stage_fire(0, 0, True)
        stage_fire(1, 1, True)

        def body2(p, _):
            c = 2 * p
            stage_proc(c, 0)

            @pl.when(c + 2 < nchunks)
            def _():
                stage_fire(c + 2, 0, False)

            stage_proc(c + 1, 1)

            @pl.when(c + 3 < nchunks)
            def _():
                stage_fire(c + 3, 1, False)

            return 0

        lax.fori_loop(0, nchunks // 2, body2, 0)
        pltpu.make_async_copy(z_hbm.at[pl.ds(0, v)], a0, so0).wait()
        pltpu.make_async_copy(z_hbm.at[pl.ds(0, v)], a1, so1).wait()

    fn = pl.kernel(
        body,
        out_type=jax.ShapeDtypeStruct((n_rows_out, co), jnp.float32),
        mesh=_sc_mesh(),
        compiler_params=_SC_PARAMS,
        scratch_types=[
            pltpu.VMEM((rows_per_w * SEQ,), jnp.int32),
            pltpu.VMEM((v, co), jnp.float32),
            pltpu.VMEM((v, co), jnp.float32),
            pltpu.VMEM((co,), jnp.float32),
            pltpu.SemaphoreType.DMA,
            pltpu.SemaphoreType.DMA,
            pltpu.SemaphoreType.DMA,
            pltpu.SemaphoreType.DMA,
        ],
    )
    return fn(zf, idxg2, bias)


def _conv_w2(w, ci, co):
    return w.reshape(SEQ, ci, co).transpose(1, 0, 2).reshape(ci, SEQ * co)


def _gather_idx(idx, n_in, v):
    b_off = (jnp.arange(BSZ, dtype=jnp.int32) * n_in)[:, None, None]
    s_off = jnp.arange(SEQ, dtype=jnp.int32)[None, None, :]
    g = (b_off + idx[None, :, :]) * SEQ + s_off
    g = g.reshape(-1, SEQ)
    return g.reshape(-1, v, SEQ).transpose(0, 2, 1).reshape(-1)


def kernel(x, spiral_idx0, spiral_idx1, spiral_idx2, spiral_idx3,
           up_rows0, up_cols0, up_vals0, up_rows1, up_cols1, up_vals1,
           up_rows2, up_cols2, up_vals2, up_rows3, up_cols3, up_vals3,
           fc_W, fc_b, Wd0, bd0, Wd1, bd1, Wd2, bd2, Wd3, bd3, Wout, bout):
    levels = [
        (NV[4], NV[3], 256, 256, spiral_idx3, up_rows3, up_cols3, up_vals3, Wd0, bd0, 32),
        (NV[3], NV[2], 256, 128, spiral_idx2, up_rows2, up_cols2, up_vals2, Wd1, bd1, 128),
        (NV[2], NV[1], 128, 64, spiral_idx1, up_rows1, up_cols1, up_vals1, Wd2, bd2, 128),
        (NV[1], NV[0], 64, 64, spiral_idx0, up_rows0, up_cols0, up_vals0, Wd3, bd3, 128),
    ]

    h = _matmul(x, fc_W, fc_b, block_m=8).reshape(BSZ * NV[4], 256)

    b_ar = jnp.arange(BSZ, dtype=jnp.int32)
    for n_in, n_out, ci, co, sp_idx, rows, cols, vals, W, bb, vch in levels:
        nnz = rows.shape[0]
        nnz_p = -(-nnz // 128) * 128
        pad = nnz_p - nnz
        if pad:
            rows = jnp.concatenate(
                [rows, jnp.full((pad,), n_out - 1, jnp.int32)])
            cols = jnp.concatenate([cols, jnp.zeros((pad,), jnp.int32)])
            vals = jnp.concatenate([vals, jnp.zeros((pad,), jnp.float32)])
        rpw = n_out // NS
        bounds = jnp.searchsorted(
            rows, jnp.arange(NS + 1, dtype=jnp.int32) * rpw).astype(jnp.int32)
        bounds = jnp.concatenate(
            [bounds, jnp.zeros((32 - NS - 1,), jnp.int32)])
        rvc = jnp.stack([
            rows.astype(jnp.float32).reshape(-1, 128),
            vals.reshape(-1, 128),
            cols.astype(jnp.float32).reshape(-1, 128),
        ], axis=1).reshape(-1)
        y = _sc_pool(h, rvc, bounds, n_in, n_out, ci)
        zf = _matmul_z(y, _conv_w2(W, ci, co), co)
        idxg2 = _gather_idx(sp_idx, n_out, vch)
        h = _sc_gsum(zf, idxg2, bb, BSZ * n_out, co, vch, elu=True)

    co_p = 16
    w2o = jnp.zeros((64, SEQ * co_p), jnp.float32)
    wr = Wout.reshape(SEQ, 64, 3).transpose(1, 0, 2)
    w2o = w2o.reshape(64, SEQ, co_p).at[:, :, :3].set(wr).reshape(64, SEQ * co_p)
    bo = jnp.zeros((co_p,), jnp.float32).at[:3].set(bout)
    zf = _matmul_z(h, w2o, co_p)
    idxg2 = _gather_idx(spiral_idx0, NV[0], 128)
    out = _sc_gsum(zf, idxg2, bo, BSZ * NV[0], co_p, 128, elu=False)
    return out[:, :3].reshape(BSZ, NV[0], 3)
